# baseline (speedup 1.0000x reference)
import jax
import jax.numpy as jnp
from jax.experimental import pallas as pl
from jax.experimental.pallas import tpu as pltpu


_PAR = 2
_GRID = 2


def _body(idx_ref, emb_ref, out_ref):
    j = pl.program_id(1)
    rs = jax.lax.dot_general(
        jnp.ones((1, emb_ref.shape[1]), jnp.float32),
        emb_ref[...],
        (((1,), (1,)), ((), ())),
        preferred_element_type=jnp.float32,
        precision=jax.lax.Precision.HIGHEST,
    )
    idx = idx_ref[...]
    table = jnp.broadcast_to(rs, (idx.shape[0], rs.shape[1]))
    vals = jnp.take_along_axis(table, idx, axis=1)
    part = jnp.sum(vals, keepdims=True).reshape(1, 1)
    pos0 = (jax.lax.broadcasted_iota(jnp.int32, (8, 128), 1)
            + jax.lax.broadcasted_iota(jnp.int32, (8, 128), 0)) == 0
    row = jnp.where(pos0, jnp.broadcast_to(part, (8, 128)), 0.0)

    @pl.when(j == 0)
    def _():
        out_ref[...] = row

    @pl.when(j > 0)
    def _():
        out_ref[...] += row


def _finish_body(parts_ref, out_ref):
    out_ref[...] = jnp.sum(parts_ref[...], keepdims=True).reshape(1, 1) - 1.0


def kernel(input_batch, emb36a, emb36b, emb48):
    del emb36a, emb36b
    n, c = input_batch.shape
    block = n // (_PAR * _GRID)
    parts = pl.pallas_call(
        _body,
        grid=(_PAR, _GRID),
        in_specs=[
            pl.BlockSpec((block, c), lambda i, j: (i * _GRID + j, 0)),
            pl.BlockSpec(emb48.shape, lambda i, j: (0, 0)),
        ],
        out_specs=pl.BlockSpec((8, 128), lambda i, j: (i, 0)),
        out_shape=jax.ShapeDtypeStruct((_PAR * 8, 128), jnp.float32),
        compiler_params=pltpu.CompilerParams(
            dimension_semantics=("parallel", "arbitrary")
        ),
    )(input_batch, emb48)
    out = pl.pallas_call(
        _finish_body,
        out_shape=jax.ShapeDtypeStruct((1, 1), jnp.float32),
    )(parts)
    return out.reshape(())
